# baseline (device time: 47163 ns/iter reference)
import jax
import jax.numpy as jnp
from jax import lax
from jax.experimental import pallas as pl
from jax.experimental.pallas import tpu as pltpu

N_DEV = 8
N_GLOBAL = 8192
EPS = 1e-5
M = 4096
H = M // 2
C = 1024
N_CHUNK = M // C


def kernel(x, gamma, beta):
    m, n_per = x.shape

    def body(x_ref, gamma_ref, beta_ref, out_ref,
             comm_ref, stage_ref, send_sems, recv_sems, out_sems):
        my_pos = lax.axis_index("i")

        barrier_sem = pltpu.get_barrier_semaphore()
        for k in range(1, N_DEV):
            pl.semaphore_signal(
                barrier_sem, inc=1,
                device_id=((my_pos + k) % N_DEV,),
                device_id_type=pl.DeviceIdType.MESH,
            )
        pl.semaphore_wait(barrier_sem, N_DEV - 1)

        g = gamma_ref[...][None, :]
        b = beta_ref[...][None, :]

        def partials(h):
            a = x_ref[pl.ds(h * H, H), :]
            s1 = jnp.sum(a, axis=1, keepdims=True)
            s2 = jnp.sum(a * a, axis=1, keepdims=True)
            comm_ref[h * N_DEV + my_pos] = jnp.transpose(
                jnp.concatenate([s1, s2], axis=1))
            return a

        def send_half(h):
            sends = []
            for k in range(1, N_DEV):
                peer = (my_pos + k) % N_DEV
                rdma = pltpu.make_async_remote_copy(
                    src_ref=comm_ref.at[h * N_DEV + my_pos],
                    dst_ref=comm_ref.at[h * N_DEV + my_pos],
                    send_sem=send_sems.at[h * N_DEV + k],
                    recv_sem=recv_sems.at[h * N_DEV + my_pos],
                    device_id=(peer,),
                    device_id_type=pl.DeviceIdType.MESH,
                )
                rdma.start()
                sends.append(rdma)
            return sends

        def wait_half(h):
            for k in range(1, N_DEV):
                src = (my_pos + k) % N_DEV
                pltpu.make_async_remote_copy(
                    src_ref=comm_ref.at[h * N_DEV + src],
                    dst_ref=comm_ref.at[h * N_DEV + src],
                    send_sem=send_sems.at[h * N_DEV + k],
                    recv_sem=recv_sems.at[h * N_DEV + src],
                    device_id=(src,),
                    device_id_type=pl.DeviceIdType.MESH,
                ).wait_recv()
            totals = jnp.sum(
                comm_ref[pl.ds(h * N_DEV, N_DEV)], axis=0)
            t = jnp.transpose(totals)
            mean = t[:, 0:1] / N_GLOBAL
            var = t[:, 1:2] / N_GLOBAL - mean * mean
            return mean, lax.rsqrt(var + EPS)

        out_dmas = [None] * N_CHUNK

        def normalize_chunk(c, a_half, mean, rstd):
            off = (c % (N_CHUNK // 2)) * C
            slot = c % 2
            if c >= 2:
                out_dmas[c - 2].wait()
            av = a_half[off:off + C, :]
            mv = mean[off:off + C, :]
            rv = rstd[off:off + C, :]
            stage_ref[slot] = (av - mv) * rv * g + b
            dma = pltpu.make_async_copy(
                stage_ref.at[slot],
                out_ref.at[pl.ds(c * C, C), :],
                out_sems.at[c % 2],
            )
            dma.start()
            out_dmas[c] = dma

        a0 = partials(0)
        sends0 = send_half(0)
        a1 = partials(1)
        sends1 = send_half(1)
        mean0, rstd0 = wait_half(0)
        normalize_chunk(0, a0, mean0, rstd0)
        normalize_chunk(1, a0, mean0, rstd0)
        mean1, rstd1 = wait_half(1)
        normalize_chunk(2, a1, mean1, rstd1)
        normalize_chunk(3, a1, mean1, rstd1)

        out_dmas[2].wait()
        out_dmas[3].wait()
        for rdma in sends0 + sends1:
            rdma.wait_send()

    return pl.pallas_call(
        body,
        out_shape=jax.ShapeDtypeStruct((m, n_per), jnp.float32),
        in_specs=[
            pl.BlockSpec(memory_space=pltpu.VMEM),
            pl.BlockSpec(memory_space=pltpu.VMEM),
            pl.BlockSpec(memory_space=pltpu.VMEM),
        ],
        out_specs=pl.BlockSpec(memory_space=pl.ANY),
        input_output_aliases={0: 0},
        scratch_shapes=[
            pltpu.VMEM((2 * N_DEV, 2, H), jnp.float32),
            pltpu.VMEM((2, C, 1024), jnp.float32),
            pltpu.SemaphoreType.DMA((2 * N_DEV,)),
            pltpu.SemaphoreType.DMA((2 * N_DEV,)),
            pltpu.SemaphoreType.DMA((2,)),
        ],
        compiler_params=pltpu.CompilerParams(
            collective_id=0,
            vmem_limit_bytes=60 * 1024 * 1024,
        ),
    )(x, gamma, beta)


# device time: 33047 ns/iter; 1.4271x vs baseline; 1.4271x over previous
import jax
import jax.numpy as jnp
from jax import lax
from jax.experimental import pallas as pl
from jax.experimental.pallas import tpu as pltpu

N_DEV = 8
N_GLOBAL = 8192
EPS = 1e-5
M = 4096
H = M // 2
CI = 512
N_IN = M // CI
C = 1024
N_CHUNK = M // C


def kernel(x, gamma, beta):
    m, n_per = x.shape

    def body(x_ref, gamma_ref, beta_ref, out_ref,
             xv_ref, comm_ref, pstats_ref, stage_ref,
             in_sems, send_sems, recv_sems, out_sems):
        my_pos = lax.axis_index("i")

        barrier_sem = pltpu.get_barrier_semaphore()
        for k in range(1, N_DEV):
            pl.semaphore_signal(
                barrier_sem, inc=1,
                device_id=((my_pos + k) % N_DEV,),
                device_id_type=pl.DeviceIdType.MESH,
            )
        pl.semaphore_wait(barrier_sem, N_DEV - 1)

        in_dmas = []
        for c in range(N_IN):
            rows = pl.ds(c * CI, CI)
            dma = pltpu.make_async_copy(
                x_ref.at[rows, :], xv_ref.at[rows, :], in_sems.at[c])
            dma.start()
            in_dmas.append(dma)

        g = gamma_ref[...][None, :]
        b = beta_ref[...][None, :]

        def partials_chunk(c):
            in_dmas[c].wait()
            a = xv_ref[pl.ds(c * CI, CI), :]
            pstats_ref[pl.ds(c * CI, CI), 0:1] = jnp.sum(a, axis=1, keepdims=True)
            pstats_ref[pl.ds(c * CI, CI), 1:2] = jnp.sum(a * a, axis=1, keepdims=True)

        def send_half(h):
            comm_ref[h * N_DEV + my_pos] = jnp.transpose(
                pstats_ref[pl.ds(h * H, H), :])
            sends = []
            for k in range(1, N_DEV):
                peer = (my_pos + k) % N_DEV
                rdma = pltpu.make_async_remote_copy(
                    src_ref=comm_ref.at[h * N_DEV + my_pos],
                    dst_ref=comm_ref.at[h * N_DEV + my_pos],
                    send_sem=send_sems.at[h * N_DEV + k],
                    recv_sem=recv_sems.at[h * N_DEV + my_pos],
                    device_id=(peer,),
                    device_id_type=pl.DeviceIdType.MESH,
                )
                rdma.start()
                sends.append(rdma)
            return sends

        def wait_half(h):
            for k in range(1, N_DEV):
                src = (my_pos + k) % N_DEV
                pltpu.make_async_remote_copy(
                    src_ref=comm_ref.at[h * N_DEV + src],
                    dst_ref=comm_ref.at[h * N_DEV + src],
                    send_sem=send_sems.at[h * N_DEV + k],
                    recv_sem=recv_sems.at[h * N_DEV + src],
                    device_id=(src,),
                    device_id_type=pl.DeviceIdType.MESH,
                ).wait_recv()
            totals = jnp.sum(
                comm_ref[pl.ds(h * N_DEV, N_DEV)], axis=0)
            t = jnp.transpose(totals)
            mean = t[:, 0:1] / N_GLOBAL
            var = t[:, 1:2] / N_GLOBAL - mean * mean
            return mean, lax.rsqrt(var + EPS)

        out_dmas = [None] * N_CHUNK

        def normalize_chunk(c, mean, rstd):
            off = (c % (N_CHUNK // 2)) * C
            slot = c % 2
            if c >= 2:
                out_dmas[c - 2].wait()
            av = xv_ref[pl.ds(c * C, C), :]
            mv = mean[off:off + C, :]
            rv = rstd[off:off + C, :]
            stage_ref[slot] = (av - mv) * rv * g + b
            dma = pltpu.make_async_copy(
                stage_ref.at[slot],
                out_ref.at[pl.ds(c * C, C), :],
                out_sems.at[c % 2],
            )
            dma.start()
            out_dmas[c] = dma

        for c in range(N_IN // 2):
            partials_chunk(c)
        sends0 = send_half(0)
        for c in range(N_IN // 2, N_IN):
            partials_chunk(c)
        sends1 = send_half(1)
        mean0, rstd0 = wait_half(0)
        normalize_chunk(0, mean0, rstd0)
        normalize_chunk(1, mean0, rstd0)
        mean1, rstd1 = wait_half(1)
        normalize_chunk(2, mean1, rstd1)
        normalize_chunk(3, mean1, rstd1)

        out_dmas[2].wait()
        out_dmas[3].wait()
        for rdma in sends0 + sends1:
            rdma.wait_send()

    return pl.pallas_call(
        body,
        out_shape=jax.ShapeDtypeStruct((m, n_per), jnp.float32),
        in_specs=[
            pl.BlockSpec(memory_space=pl.ANY),
            pl.BlockSpec(memory_space=pltpu.VMEM),
            pl.BlockSpec(memory_space=pltpu.VMEM),
        ],
        out_specs=pl.BlockSpec(memory_space=pl.ANY),
        scratch_shapes=[
            pltpu.VMEM((M, 1024), jnp.float32),
            pltpu.VMEM((2 * N_DEV, 2, H), jnp.float32),
            pltpu.VMEM((M, 2), jnp.float32),
            pltpu.VMEM((2, C, 1024), jnp.float32),
            pltpu.SemaphoreType.DMA((N_IN,)),
            pltpu.SemaphoreType.DMA((2 * N_DEV,)),
            pltpu.SemaphoreType.DMA((2 * N_DEV,)),
            pltpu.SemaphoreType.DMA((2,)),
        ],
        compiler_params=pltpu.CompilerParams(
            collective_id=0,
            vmem_limit_bytes=60 * 1024 * 1024,
        ),
    )(x, gamma, beta)


# device time: 32749 ns/iter; 1.4401x vs baseline; 1.0091x over previous
import jax
import jax.numpy as jnp
from jax import lax
from jax.experimental import pallas as pl
from jax.experimental.pallas import tpu as pltpu

N_DEV = 8
N_GLOBAL = 8192
EPS = 1e-5
M = 4096
H = M // 2
CI = 512
N_IN = M // CI
C = 1024
N_CHUNK = M // C


def kernel(x, gamma, beta):
    m, n_per = x.shape

    def body(x_ref, gamma_ref, beta_ref, out_ref,
             xv_ref, comm_ref, stage_ref,
             in_sems, send_sems, recv_sems, out_sems):
        my_pos = lax.axis_index("i")

        barrier_sem = pltpu.get_barrier_semaphore()
        for k in range(1, N_DEV):
            pl.semaphore_signal(
                barrier_sem, inc=1,
                device_id=((my_pos + k) % N_DEV,),
                device_id_type=pl.DeviceIdType.MESH,
            )
        pl.semaphore_wait(barrier_sem, N_DEV - 1)

        in_dmas = []
        for c in range(N_IN):
            rows = pl.ds(c * CI, CI)
            dma = pltpu.make_async_copy(
                x_ref.at[rows, :], xv_ref.at[rows, :], in_sems.at[c])
            dma.start()
            in_dmas.append(dma)

        g = gamma_ref[...][None, :]
        b = beta_ref[...][None, :]

        def partials_chunk(c):
            in_dmas[c].wait()
            a = xv_ref[pl.ds(c * CI, CI), :]
            st = jnp.stack([jnp.sum(a, axis=1), jnp.sum(a * a, axis=1)])
            off = (c % (N_IN // 2)) * CI
            comm_ref[(c // (N_IN // 2)) * N_DEV + my_pos, :, off:off + CI] = st

        def send_half(h):
            sends = []
            for k in range(1, N_DEV):
                peer = (my_pos + k) % N_DEV
                rdma = pltpu.make_async_remote_copy(
                    src_ref=comm_ref.at[h * N_DEV + my_pos],
                    dst_ref=comm_ref.at[h * N_DEV + my_pos],
                    send_sem=send_sems.at[h * N_DEV + k],
                    recv_sem=recv_sems.at[h * N_DEV + my_pos],
                    device_id=(peer,),
                    device_id_type=pl.DeviceIdType.MESH,
                )
                rdma.start()
                sends.append(rdma)
            return sends

        def wait_half(h):
            for k in range(1, N_DEV):
                src = (my_pos + k) % N_DEV
                pltpu.make_async_remote_copy(
                    src_ref=comm_ref.at[h * N_DEV + src],
                    dst_ref=comm_ref.at[h * N_DEV + src],
                    send_sem=send_sems.at[h * N_DEV + k],
                    recv_sem=recv_sems.at[h * N_DEV + src],
                    device_id=(src,),
                    device_id_type=pl.DeviceIdType.MESH,
                ).wait_recv()
            totals = jnp.sum(
                comm_ref[pl.ds(h * N_DEV, N_DEV)], axis=0)
            t = jnp.transpose(totals)
            mean = t[:, 0:1] / N_GLOBAL
            var = t[:, 1:2] / N_GLOBAL - mean * mean
            return mean, lax.rsqrt(var + EPS)

        out_dmas = [None] * N_CHUNK

        def normalize_chunk(c, mean, rstd):
            off = (c % (N_CHUNK // 2)) * C
            slot = c % 2
            if c >= 2:
                out_dmas[c - 2].wait()
            av = xv_ref[pl.ds(c * C, C), :]
            mv = mean[off:off + C, :]
            rv = rstd[off:off + C, :]
            stage_ref[slot] = (av - mv) * rv * g + b
            dma = pltpu.make_async_copy(
                stage_ref.at[slot],
                out_ref.at[pl.ds(c * C, C), :],
                out_sems.at[c % 2],
            )
            dma.start()
            out_dmas[c] = dma

        for c in range(N_IN // 2):
            partials_chunk(c)
        sends0 = send_half(0)
        for c in range(N_IN // 2, N_IN):
            partials_chunk(c)
        sends1 = send_half(1)
        mean0, rstd0 = wait_half(0)
        normalize_chunk(0, mean0, rstd0)
        normalize_chunk(1, mean0, rstd0)
        mean1, rstd1 = wait_half(1)
        normalize_chunk(2, mean1, rstd1)
        normalize_chunk(3, mean1, rstd1)

        out_dmas[2].wait()
        out_dmas[3].wait()
        for rdma in sends0 + sends1:
            rdma.wait_send()

    return pl.pallas_call(
        body,
        out_shape=jax.ShapeDtypeStruct((m, n_per), jnp.float32),
        in_specs=[
            pl.BlockSpec(memory_space=pl.ANY),
            pl.BlockSpec(memory_space=pltpu.VMEM),
            pl.BlockSpec(memory_space=pltpu.VMEM),
        ],
        out_specs=pl.BlockSpec(memory_space=pl.ANY),
        scratch_shapes=[
            pltpu.VMEM((M, 1024), jnp.float32),
            pltpu.VMEM((2 * N_DEV, 2, H), jnp.float32),
            pltpu.VMEM((2, C, 1024), jnp.float32),
            pltpu.SemaphoreType.DMA((N_IN,)),
            pltpu.SemaphoreType.DMA((2 * N_DEV,)),
            pltpu.SemaphoreType.DMA((2 * N_DEV,)),
            pltpu.SemaphoreType.DMA((2,)),
        ],
        compiler_params=pltpu.CompilerParams(
            collective_id=0,
            vmem_limit_bytes=60 * 1024 * 1024,
        ),
    )(x, gamma, beta)


# device time: 32616 ns/iter; 1.4460x vs baseline; 1.0041x over previous
import jax
import jax.numpy as jnp
from jax import lax
from jax.experimental import pallas as pl
from jax.experimental.pallas import tpu as pltpu

N_DEV = 8
N_GLOBAL = 8192
EPS = 1e-5
M = 4096
H = M // 2
CI = 512
N_IN = M // CI
C = 512
N_CHUNK = M // C


def kernel(x, gamma, beta):
    m, n_per = x.shape

    def body(x_ref, gamma_ref, beta_ref, out_ref,
             xv_ref, comm_ref, stage_ref,
             in_sems, send_sems, recv_sems, out_sems):
        my_pos = lax.axis_index("i")

        in_dmas = []
        for c in range(N_IN):
            rows = pl.ds(c * CI, CI)
            dma = pltpu.make_async_copy(
                x_ref.at[rows, :], xv_ref.at[rows, :], in_sems.at[c])
            dma.start()
            in_dmas.append(dma)

        barrier_sem = pltpu.get_barrier_semaphore()
        for k in range(1, N_DEV):
            pl.semaphore_signal(
                barrier_sem, inc=1,
                device_id=((my_pos + k) % N_DEV,),
                device_id_type=pl.DeviceIdType.MESH,
            )

        g = gamma_ref[...][None, :]
        b = beta_ref[...][None, :]

        def partials_chunk(c):
            in_dmas[c].wait()
            a = xv_ref[pl.ds(c * CI, CI), :]
            st = jnp.stack([jnp.sum(a, axis=1), jnp.sum(a * a, axis=1)])
            off = (c % (N_IN // 2)) * CI
            comm_ref[(c // (N_IN // 2)) * N_DEV + my_pos, :, off:off + CI] = st

        def send_half(h):
            sends = []
            for k in range(1, N_DEV):
                peer = (my_pos + k) % N_DEV
                rdma = pltpu.make_async_remote_copy(
                    src_ref=comm_ref.at[h * N_DEV + my_pos],
                    dst_ref=comm_ref.at[h * N_DEV + my_pos],
                    send_sem=send_sems.at[h * N_DEV + k],
                    recv_sem=recv_sems.at[h * N_DEV + my_pos],
                    device_id=(peer,),
                    device_id_type=pl.DeviceIdType.MESH,
                )
                rdma.start()
                sends.append(rdma)
            return sends

        def wait_half(h):
            for k in range(1, N_DEV):
                src = (my_pos + k) % N_DEV
                pltpu.make_async_remote_copy(
                    src_ref=comm_ref.at[h * N_DEV + src],
                    dst_ref=comm_ref.at[h * N_DEV + src],
                    send_sem=send_sems.at[h * N_DEV + k],
                    recv_sem=recv_sems.at[h * N_DEV + src],
                    device_id=(src,),
                    device_id_type=pl.DeviceIdType.MESH,
                ).wait_recv()
            totals = jnp.sum(
                comm_ref[pl.ds(h * N_DEV, N_DEV)], axis=0)
            t = jnp.transpose(totals)
            mean = t[:, 0:1] / N_GLOBAL
            var = t[:, 1:2] / N_GLOBAL - mean * mean
            return mean, lax.rsqrt(var + EPS)

        out_dmas = [None] * N_CHUNK

        def normalize_chunk(c, mean, rstd):
            off = (c % (N_CHUNK // 2)) * C
            slot = c % 2
            if c >= 2:
                out_dmas[c - 2].wait()
            av = xv_ref[pl.ds(c * C, C), :]
            mv = mean[off:off + C, :]
            rv = rstd[off:off + C, :]
            stage_ref[slot] = (av - mv) * rv * g + b
            dma = pltpu.make_async_copy(
                stage_ref.at[slot],
                out_ref.at[pl.ds(c * C, C), :],
                out_sems.at[c % 2],
            )
            dma.start()
            out_dmas[c] = dma

        for c in range(N_IN // 2):
            partials_chunk(c)
        pl.semaphore_wait(barrier_sem, N_DEV - 1)
        sends0 = send_half(0)
        for c in range(N_IN // 2, N_IN):
            partials_chunk(c)
        sends1 = send_half(1)
        mean0, rstd0 = wait_half(0)
        for c in range(N_CHUNK // 2):
            normalize_chunk(c, mean0, rstd0)
        mean1, rstd1 = wait_half(1)
        for c in range(N_CHUNK // 2, N_CHUNK):
            normalize_chunk(c, mean1, rstd1)

        out_dmas[N_CHUNK - 2].wait()
        out_dmas[N_CHUNK - 1].wait()
        for rdma in sends0 + sends1:
            rdma.wait_send()

    return pl.pallas_call(
        body,
        out_shape=jax.ShapeDtypeStruct((m, n_per), jnp.float32),
        in_specs=[
            pl.BlockSpec(memory_space=pl.ANY),
            pl.BlockSpec(memory_space=pltpu.VMEM),
            pl.BlockSpec(memory_space=pltpu.VMEM),
        ],
        out_specs=pl.BlockSpec(memory_space=pl.ANY),
        scratch_shapes=[
            pltpu.VMEM((M, 1024), jnp.float32),
            pltpu.VMEM((2 * N_DEV, 2, H), jnp.float32),
            pltpu.VMEM((2, C, 1024), jnp.float32),
            pltpu.SemaphoreType.DMA((N_IN,)),
            pltpu.SemaphoreType.DMA((2 * N_DEV,)),
            pltpu.SemaphoreType.DMA((2 * N_DEV,)),
            pltpu.SemaphoreType.DMA((2,)),
        ],
        compiler_params=pltpu.CompilerParams(
            collective_id=0,
            vmem_limit_bytes=60 * 1024 * 1024,
        ),
    )(x, gamma, beta)


# device time: 29615 ns/iter; 1.5925x vs baseline; 1.1013x over previous
import jax
import jax.numpy as jnp
from jax import lax
from jax.experimental import pallas as pl
from jax.experimental.pallas import tpu as pltpu

N_DEV = 8
N_GLOBAL = 8192
EPS = 1e-5
M = 4096
H = M // 2
CI = 512
N_IN = M // CI
BM = 512
NB = M // BM


def _stats_call(x):

    def body(x_ref, mr_ref, xv_ref, comm_ref, in_sems, send_sems, recv_sems):
        my_pos = lax.axis_index("i")

        in_dmas = []
        for c in range(N_IN):
            rows = pl.ds(c * CI, CI)
            dma = pltpu.make_async_copy(
                x_ref.at[rows, :], xv_ref.at[rows, :], in_sems.at[c])
            dma.start()
            in_dmas.append(dma)

        barrier_sem = pltpu.get_barrier_semaphore()
        for k in range(1, N_DEV):
            pl.semaphore_signal(
                barrier_sem, inc=1,
                device_id=((my_pos + k) % N_DEV,),
                device_id_type=pl.DeviceIdType.MESH,
            )

        def partials_chunk(c):
            in_dmas[c].wait()
            a = xv_ref[pl.ds(c * CI, CI), :]
            st = jnp.stack([jnp.sum(a, axis=1), jnp.sum(a * a, axis=1)])
            off = (c % (N_IN // 2)) * CI
            comm_ref[(c // (N_IN // 2)) * N_DEV + my_pos, :, off:off + CI] = st

        def send_half(h):
            sends = []
            for k in range(1, N_DEV):
                peer = (my_pos + k) % N_DEV
                rdma = pltpu.make_async_remote_copy(
                    src_ref=comm_ref.at[h * N_DEV + my_pos],
                    dst_ref=comm_ref.at[h * N_DEV + my_pos],
                    send_sem=send_sems.at[h * N_DEV + k],
                    recv_sem=recv_sems.at[h * N_DEV + my_pos],
                    device_id=(peer,),
                    device_id_type=pl.DeviceIdType.MESH,
                )
                rdma.start()
                sends.append(rdma)
            return sends

        def wait_half(h):
            for k in range(1, N_DEV):
                src = (my_pos + k) % N_DEV
                pltpu.make_async_remote_copy(
                    src_ref=comm_ref.at[h * N_DEV + src],
                    dst_ref=comm_ref.at[h * N_DEV + src],
                    send_sem=send_sems.at[h * N_DEV + k],
                    recv_sem=recv_sems.at[h * N_DEV + src],
                    device_id=(src,),
                    device_id_type=pl.DeviceIdType.MESH,
                ).wait_recv()
            totals = jnp.sum(
                comm_ref[pl.ds(h * N_DEV, N_DEV)], axis=0)
            t = jnp.transpose(totals)
            mean = t[:, 0:1] / N_GLOBAL
            var = t[:, 1:2] / N_GLOBAL - mean * mean
            mr_ref[pl.ds(h * H, H), 0:1] = mean
            mr_ref[pl.ds(h * H, H), 1:2] = lax.rsqrt(var + EPS)

        for c in range(N_IN // 2):
            partials_chunk(c)
        pl.semaphore_wait(barrier_sem, N_DEV - 1)
        sends0 = send_half(0)
        for c in range(N_IN // 2, N_IN):
            partials_chunk(c)
        sends1 = send_half(1)
        wait_half(0)
        wait_half(1)
        for rdma in sends0 + sends1:
            rdma.wait_send()

    return pl.pallas_call(
        body,
        out_shape=jax.ShapeDtypeStruct((M, 2), jnp.float32),
        in_specs=[pl.BlockSpec(memory_space=pl.ANY)],
        out_specs=pl.BlockSpec(memory_space=pltpu.VMEM),
        scratch_shapes=[
            pltpu.VMEM((M, 1024), jnp.float32),
            pltpu.VMEM((2 * N_DEV, 2, H), jnp.float32),
            pltpu.SemaphoreType.DMA((N_IN,)),
            pltpu.SemaphoreType.DMA((2 * N_DEV,)),
            pltpu.SemaphoreType.DMA((2 * N_DEV,)),
        ],
        compiler_params=pltpu.CompilerParams(
            collective_id=0,
            vmem_limit_bytes=60 * 1024 * 1024,
        ),
    )(x)


def _normalize_call(x, mr, gamma, beta):
    m, n_per = x.shape

    def body(x_ref, mr_ref, gamma_ref, beta_ref, out_ref):
        s = pl.program_id(0)
        a = x_ref[...]
        mean = mr_ref[pl.ds(s * BM, BM), 0:1]
        rstd = mr_ref[pl.ds(s * BM, BM), 1:2]
        out_ref[...] = ((a - mean) * rstd * gamma_ref[...][None, :]
                        + beta_ref[...][None, :])

    return pl.pallas_call(
        body,
        grid=(NB,),
        out_shape=jax.ShapeDtypeStruct((m, n_per), jnp.float32),
        in_specs=[
            pl.BlockSpec((BM, n_per), lambda s: (s, 0)),
            pl.BlockSpec(memory_space=pltpu.VMEM),
            pl.BlockSpec((n_per,), lambda s: (0,)),
            pl.BlockSpec((n_per,), lambda s: (0,)),
        ],
        out_specs=pl.BlockSpec((BM, n_per), lambda s: (s, 0)),
        compiler_params=pltpu.CompilerParams(
            vmem_limit_bytes=60 * 1024 * 1024,
        ),
    )(x, mr, gamma, beta)


def kernel(x, gamma, beta):
    mr = _stats_call(x)
    return _normalize_call(x, mr, gamma, beta)


# device time: 28657 ns/iter; 1.6458x vs baseline; 1.0334x over previous
import jax
import jax.numpy as jnp
from jax import lax
from jax.experimental import pallas as pl
from jax.experimental.pallas import tpu as pltpu

N_DEV = 8
N_GLOBAL = 8192
EPS = 1e-5
M = 4096
H = M // 2
CI = 512
N_IN = M // CI
BM = 1024
NB = M // BM


def _stats_call(x):

    def body(x_ref, mr_ref, xv_ref, comm_ref, in_sems, send_sems, recv_sems):
        my_pos = lax.axis_index("i")

        in_dmas = []
        for c in range(N_IN):
            rows = pl.ds(c * CI, CI)
            dma = pltpu.make_async_copy(
                x_ref.at[rows, :], xv_ref.at[rows, :], in_sems.at[c])
            dma.start()
            in_dmas.append(dma)

        barrier_sem = pltpu.get_barrier_semaphore()
        for k in range(1, N_DEV):
            pl.semaphore_signal(
                barrier_sem, inc=1,
                device_id=((my_pos + k) % N_DEV,),
                device_id_type=pl.DeviceIdType.MESH,
            )

        def partials_chunk(c):
            in_dmas[c].wait()
            a = xv_ref[pl.ds(c * CI, CI), :]
            st = jnp.stack([jnp.sum(a, axis=1), jnp.sum(a * a, axis=1)])
            off = (c % (N_IN // 2)) * CI
            comm_ref[(c // (N_IN // 2)) * N_DEV + my_pos, :, off:off + CI] = st

        def send_half(h):
            sends = []
            for k in range(1, N_DEV):
                peer = (my_pos + k) % N_DEV
                rdma = pltpu.make_async_remote_copy(
                    src_ref=comm_ref.at[h * N_DEV + my_pos],
                    dst_ref=comm_ref.at[h * N_DEV + my_pos],
                    send_sem=send_sems.at[h * N_DEV + k],
                    recv_sem=recv_sems.at[h * N_DEV + my_pos],
                    device_id=(peer,),
                    device_id_type=pl.DeviceIdType.MESH,
                )
                rdma.start()
                sends.append(rdma)
            return sends

        def wait_half(h):
            for k in range(1, N_DEV):
                src = (my_pos + k) % N_DEV
                pltpu.make_async_remote_copy(
                    src_ref=comm_ref.at[h * N_DEV + src],
                    dst_ref=comm_ref.at[h * N_DEV + src],
                    send_sem=send_sems.at[h * N_DEV + k],
                    recv_sem=recv_sems.at[h * N_DEV + src],
                    device_id=(src,),
                    device_id_type=pl.DeviceIdType.MESH,
                ).wait_recv()
            totals = jnp.sum(
                comm_ref[pl.ds(h * N_DEV, N_DEV)], axis=0)
            t = jnp.transpose(totals)
            mean = t[:, 0:1] / N_GLOBAL
            var = t[:, 1:2] / N_GLOBAL - mean * mean
            mr_ref[pl.ds(h * H, H), 0:1] = mean
            mr_ref[pl.ds(h * H, H), 1:2] = lax.rsqrt(var + EPS)

        for c in range(N_IN // 2):
            partials_chunk(c)
        pl.semaphore_wait(barrier_sem, N_DEV - 1)
        sends0 = send_half(0)
        for c in range(N_IN // 2, N_IN):
            partials_chunk(c)
        sends1 = send_half(1)
        wait_half(0)
        wait_half(1)
        for rdma in sends0 + sends1:
            rdma.wait_send()

    return pl.pallas_call(
        body,
        out_shape=jax.ShapeDtypeStruct((M, 2), jnp.float32),
        in_specs=[pl.BlockSpec(memory_space=pl.ANY)],
        out_specs=pl.BlockSpec(memory_space=pltpu.VMEM),
        scratch_shapes=[
            pltpu.VMEM((M, 1024), jnp.float32),
            pltpu.VMEM((2 * N_DEV, 2, H), jnp.float32),
            pltpu.SemaphoreType.DMA((N_IN,)),
            pltpu.SemaphoreType.DMA((2 * N_DEV,)),
            pltpu.SemaphoreType.DMA((2 * N_DEV,)),
        ],
        compiler_params=pltpu.CompilerParams(
            collective_id=0,
            vmem_limit_bytes=60 * 1024 * 1024,
        ),
    )(x)


def _normalize_call(x, mr, gamma, beta):
    m, n_per = x.shape

    def body(x_ref, mr_ref, gamma_ref, beta_ref, out_ref):
        a = x_ref[...]
        mean = mr_ref[:, 0:1]
        rstd = mr_ref[:, 1:2]
        out_ref[...] = ((a - mean) * rstd * gamma_ref[...][None, :]
                        + beta_ref[...][None, :])

    return pl.pallas_call(
        body,
        grid=(NB,),
        out_shape=jax.ShapeDtypeStruct((m, n_per), jnp.float32),
        in_specs=[
            pl.BlockSpec((BM, n_per), lambda s: (s, 0)),
            pl.BlockSpec((BM, 2), lambda s: (s, 0)),
            pl.BlockSpec((n_per,), lambda s: (0,)),
            pl.BlockSpec((n_per,), lambda s: (0,)),
        ],
        out_specs=pl.BlockSpec((BM, n_per), lambda s: (s, 0)),
        compiler_params=pltpu.CompilerParams(
            dimension_semantics=("parallel",),
            vmem_limit_bytes=60 * 1024 * 1024,
        ),
    )(x, mr, gamma, beta)


def kernel(x, gamma, beta):
    mr = _stats_call(x)
    return _normalize_call(x, mr, gamma, beta)
